# baseline (speedup 1.0000x reference)
import jax
import jax.numpy as jnp
from jax.experimental import pallas as pl
from jax.experimental.pallas import tpu as pltpu


def _body(x_hbm, o_ref, buf, sems):
    copies = [
        pltpu.make_async_copy(
            x_hbm.at[:, pl.ds(2048 * i, 128)],
            buf.at[i],
            sems.at[i % 8],
        )
        for i in range(32)
    ]
    for cp in copies:
        cp.start()
    for cp in copies:
        cp.wait()
    o_ref[...] = jnp.concatenate([buf[i, :, 0:1] for i in range(32)], axis=1)


def kernel(x):
    return pl.pallas_call(
        _body,
        in_specs=[pl.BlockSpec(memory_space=pltpu.MemorySpace.HBM)],
        out_specs=pl.BlockSpec(memory_space=pltpu.MemorySpace.VMEM),
        out_shape=jax.ShapeDtypeStruct((1024, 32), jnp.float32),
        scratch_shapes=[
            pltpu.VMEM((32, 1024, 128), jnp.float32),
            pltpu.SemaphoreType.DMA((8,)),
        ],
    )(x)


# speedup vs baseline: 1.0031x; 1.0031x over previous
import jax
import jax.numpy as jnp
from jax.experimental import pallas as pl


def _body(*refs):
    o_ref = refs[-1]
    o_ref[...] = jnp.concatenate([r[:, 0:1] for r in refs[:-1]], axis=1)


def _spec(i):
    return pl.BlockSpec((1024, 128), lambda _, i=i: (0, 16 * i))


def kernel(x):
    return pl.pallas_call(
        _body,
        grid=(1,),
        in_specs=[_spec(i) for i in range(32)],
        out_specs=pl.BlockSpec((1024, 32), lambda _: (0, 0)),
        out_shape=jax.ShapeDtypeStruct((1024, 32), jnp.float32),
    )(*([x] * 32))
